# baseline (speedup 1.0000x reference)
import jax
import jax.numpy as jnp
from jax import lax
from jax.experimental import pallas as pl
from jax.experimental.pallas import tpu as pltpu
from jax.experimental.pallas import tpu_sc as plsc

N_USERS = 25000
N_ITEMS = 25000
N_TOTAL = N_USERS + N_ITEMS
HALF_NODES = N_TOTAL // 2
DIM = 64
N_EDGES = 800000
N_TILES = 16

EDGES_PER_TILE = 50176
E_PAD = N_TILES * EDGES_PER_TILE

FLUSH = 1792
CAP = FLUSH + 16
SCAN_CHUNKS = EDGES_PER_TILE // FLUSH
RBLOCKS = SCAN_CHUNKS + 1
RCAP = RBLOCKS * FLUSH

GROUPS_PER_CHUNK = FLUSH // 64
PAIRS_PER_CHUNK = GROUPS_PER_CHUNK // 2

RR_ROWS = 50
RR_CHUNKS = HALF_NODES // RR_ROWS
RR_ITERS = (RR_CHUNKS + N_TILES - 1) // N_TILES

_GATHER_DNUMS = lax.GatherDimensionNumbers(
    offset_dims=(), collapsed_slice_dims=(0,), start_index_map=(0,))


def _splat(vec16, j):
    idx = jnp.full((16, 1), j, jnp.int32)
    return lax.gather(vec16, idx, _GATHER_DNUMS, slice_sizes=(1,),
                      mode=lax.GatherScatterMode.PROMISE_IN_BOUNDS)


def _axes():
    return lax.axis_index("c"), lax.axis_index("s")


def _fill_zero(zero_v):
    def zf(i, _):
        for h in (0, 16, 32, 48):
            zero_v[i, pl.ds(h, 16)] = jnp.zeros((16,), jnp.float32)
        return 0
    lax.fori_loop(0, RR_ROWS, zf, 0)


def _zero_acc(s, acc, zero_v):
    def zb(k, _):
        ch = s + k * N_TILES
        @pl.when(ch < RR_CHUNKS)
        def _():
            pltpu.sync_copy(zero_v, acc.at[pl.ds(ch * RR_ROWS, RR_ROWS)])
        return 0
    lax.fori_loop(0, RR_ITERS, zb, 0)


def _weight_mul(rows_v, w_c, g):
    def e_body(e16, _):
        w16 = w_c[pl.ds(g * 64 + e16 * 16, 16)]
        for j in range(16):
            wj = _splat(w16, j)
            e = e16 * 16 + j
            for h in (0, 16, 32, 48):
                rows_v[e, pl.ds(h, 16)] = rows_v[e, pl.ds(h, 16)] * wj
        return 0
    lax.fori_loop(0, 4, e_body, 0)


def _do_layer(c, s, src_h, nch, pcol, prow, pw, col_c, row_c, w_c,
              rows_a, rows_b, acc, gsem_a, gsem_b, ssem_a, ssem_b):
    def chunk_body(ch, _):
        off = pl.multiple_of(ch * FLUSH, 8)
        pltpu.sync_copy(pcol.at[c, s, pl.ds(off, FLUSH)], col_c)
        pltpu.sync_copy(prow.at[c, s, pl.ds(off, FLUSH)], row_c)
        pltpu.sync_copy(pw.at[c, s, pl.ds(off, FLUSH)], w_c)

        def pair_body(i, _):
            g0 = i * 2
            dA = pltpu.async_copy(src_h.at[col_c.at[pl.ds(g0 * 64, 64)]], rows_a, gsem_a)
            dB = pltpu.async_copy(src_h.at[col_c.at[pl.ds(g0 * 64 + 64, 64)]], rows_b, gsem_b)
            dA.wait()
            _weight_mul(rows_a, w_c, g0)
            sA = pltpu.async_copy(rows_a, acc.at[row_c.at[pl.ds(g0 * 64, 64)]], ssem_a, add=True)
            dB.wait()
            _weight_mul(rows_b, w_c, g0 + 1)
            sB = pltpu.async_copy(rows_b, acc.at[row_c.at[pl.ds(g0 * 64 + 64, 64)]], ssem_b, add=True)
            sA.wait()
            sB.wait()
            return 0
        lax.fori_loop(0, PAIRS_PER_CHUNK, pair_body, 0)
        return 0
    lax.fori_loop(0, nch, chunk_body, 0)


def _body1(x_hbm, col_hbm, row_hbm, w_hbm, y_hbm, pcol, prow, pw, ktb,
           col_c, row_c, w_c, acol, arow, aw, rows_a, rows_b, zero_v, ktv,
           acc, gsem_a, gsem_b, ssem_a, ssem_b):
    c, s = _axes()
    lo = c * HALF_NODES

    zi = jnp.zeros((16,), jnp.int32)
    zf = jnp.zeros((16,), jnp.float32)

    def chunk_body(ch, carry):
        pltpu.sync_copy(col_hbm.at[s, pl.ds(ch * FLUSH, FLUSH)], col_c)
        pltpu.sync_copy(row_hbm.at[s, pl.ds(ch * FLUSH, FLUSH)], row_c)
        pltpu.sync_copy(w_hbm.at[s, pl.ds(ch * FLUSH, FLUSH)], w_c)

        def v_body(v, carry2):
            fill, op = carry2
            col16 = col_c[pl.ds(v * 16, 16)]
            row16 = row_c[pl.ds(v * 16, 16)]
            w16 = w_c[pl.ds(v * 16, 16)]
            rloc = row16 - lo
            m = (rloc >= 0) & (rloc < HALF_NODES)
            keyv = jnp.where(m, jnp.zeros((16,), jnp.int32),
                             jnp.ones((16,), jnp.int32))
            perm = plsc.sort_key_val(keyv, lax.iota(jnp.int32, 16))[1]
            pidx = jnp.reshape(perm, (16, 1))
            colc = lax.gather(col16, pidx, _GATHER_DNUMS, slice_sizes=(1,),
                              mode=lax.GatherScatterMode.PROMISE_IN_BOUNDS)
            rowc = lax.gather(rloc, pidx, _GATHER_DNUMS, slice_sizes=(1,),
                              mode=lax.GatherScatterMode.PROMISE_IN_BOUNDS)
            wc = lax.gather(w16, pidx, _GATHER_DNUMS, slice_sizes=(1,),
                            mode=lax.GatherScatterMode.PROMISE_IN_BOUNDS)
            acol[pl.ds(fill, 16)] = colc
            arow[pl.ds(fill, 16)] = rowc
            aw[pl.ds(fill, 16)] = wc
            cnt = jnp.max(plsc.all_reduce_population_count(m))
            fill = fill + cnt
            do_flush = fill >= FLUSH

            @pl.when(do_flush)
            def _():
                opa = pl.multiple_of(op, 8)
                pltpu.sync_copy(acol.at[pl.ds(0, FLUSH)], pcol.at[c, s, pl.ds(opa, FLUSH)])
                pltpu.sync_copy(arow.at[pl.ds(0, FLUSH)], prow.at[c, s, pl.ds(opa, FLUSH)])
                pltpu.sync_copy(aw.at[pl.ds(0, FLUSH)], pw.at[c, s, pl.ds(opa, FLUSH)])
                acol[pl.ds(0, 16)] = acol[pl.ds(FLUSH, 16)]
                arow[pl.ds(0, 16)] = arow[pl.ds(FLUSH, 16)]
                aw[pl.ds(0, 16)] = aw[pl.ds(FLUSH, 16)]

            fill = jnp.where(do_flush, fill - FLUSH, fill)
            op = jnp.where(do_flush, op + FLUSH, op)
            return (fill, op)
        return lax.fori_loop(0, FLUSH // 16, v_body, carry)

    fill, op = lax.fori_loop(0, SCAN_CHUNKS, chunk_body,
                             (jnp.int32(0), jnp.int32(0)))

    def z_body(k, _):
        pos = fill + k * 16
        @pl.when(pos < FLUSH)
        def _():
            acol[pl.ds(pos, 16)] = zi
            arow[pl.ds(pos, 16)] = zi
            aw[pl.ds(pos, 16)] = zf
        return 0
    lax.fori_loop(0, FLUSH // 16, z_body, 0)
    opa = pl.multiple_of(op, 8)
    pltpu.sync_copy(acol.at[pl.ds(0, FLUSH)], pcol.at[c, s, pl.ds(opa, FLUSH)])
    pltpu.sync_copy(arow.at[pl.ds(0, FLUSH)], prow.at[c, s, pl.ds(opa, FLUSH)])
    pltpu.sync_copy(aw.at[pl.ds(0, FLUSH)], pw.at[c, s, pl.ds(opa, FLUSH)])
    kt = op + fill
    ktv[pl.ds(0, 16)] = jnp.full((16,), 1, jnp.int32) * kt
    pltpu.sync_copy(ktv, ktb.at[c, s])

    _fill_zero(zero_v)
    _zero_acc(s, acc, zero_v)
    plsc.subcore_barrier()
    nch = lax.div(kt + (FLUSH - 1), FLUSH)
    _do_layer(c, s, x_hbm, nch, pcol, prow, pw, col_c, row_c, w_c,
              rows_a, rows_b, acc, gsem_a, gsem_b, ssem_a, ssem_b)
    plsc.subcore_barrier()

    def yw(k, _):
        ch = s + k * N_TILES
        @pl.when(ch < RR_CHUNKS)
        def _():
            pltpu.sync_copy(acc.at[pl.ds(ch * RR_ROWS, RR_ROWS)],
                            y_hbm.at[pl.ds(lo + ch * RR_ROWS, RR_ROWS)])
        return 0
    lax.fori_loop(0, RR_ITERS, yw, 0)


def _body2(x_hbm, y_hbm, pcol, prow, pw, ktb, out_hbm,
           col_c, row_c, w_c, rows_a, rows_b, zero_v, a_v, b_v, ktv,
           acc, gsem_a, gsem_b, ssem_a, ssem_b):
    c, s = _axes()
    lo = c * HALF_NODES
    c_v = zero_v

    pltpu.sync_copy(ktb.at[c, s], ktv)
    kt = jnp.max(ktv[pl.ds(0, 16)])
    nch = lax.div(kt + (FLUSH - 1), FLUSH)

    _fill_zero(zero_v)
    _zero_acc(s, acc, zero_v)
    plsc.subcore_barrier()
    _do_layer(c, s, y_hbm, nch, pcol, prow, pw, col_c, row_c, w_c,
              rows_a, rows_b, acc, gsem_a, gsem_b, ssem_a, ssem_b)
    plsc.subcore_barrier()

    third = jnp.float32(1.0 / 3.0)

    def epi(k, _):
        ch = s + k * N_TILES
        @pl.when(ch < RR_CHUNKS)
        def _():
            r_loc = ch * RR_ROWS
            r_glob = lo + r_loc
            pltpu.sync_copy(x_hbm.at[pl.ds(r_glob, RR_ROWS)], a_v)
            pltpu.sync_copy(y_hbm.at[pl.ds(r_glob, RR_ROWS)], b_v)
            pltpu.sync_copy(acc.at[pl.ds(r_loc, RR_ROWS)], c_v)

            def erow(r, _):
                for h in (0, 16, 32, 48):
                    a_v[r, pl.ds(h, 16)] = (
                        a_v[r, pl.ds(h, 16)] + b_v[r, pl.ds(h, 16)]
                        + c_v[r, pl.ds(h, 16)]
                    ) * third
                return 0
            lax.fori_loop(0, RR_ROWS, erow, 0)
            pltpu.sync_copy(a_v, out_hbm.at[pl.ds(r_glob, RR_ROWS)])
        return 0
    lax.fori_loop(0, RR_ITERS, epi, 0)


@jax.jit
def _run(x, colp, rowp, wp):
    mesh = plsc.VectorSubcoreMesh(core_axis_name="c", subcore_axis_name="s")
    cp = pltpu.CompilerParams(use_tc_tiling_on_sc=False, needs_layout_passes=False)
    f1 = pl.kernel(
        _body1,
        out_type=(
            jax.ShapeDtypeStruct((N_TOTAL, DIM), jnp.float32),
            jax.ShapeDtypeStruct((2, N_TILES, RCAP), jnp.int32),
            jax.ShapeDtypeStruct((2, N_TILES, RCAP), jnp.int32),
            jax.ShapeDtypeStruct((2, N_TILES, RCAP), jnp.float32),
            jax.ShapeDtypeStruct((2, N_TILES, 16), jnp.int32),
        ),
        mesh=mesh,
        scratch_types=[
            pltpu.VMEM((FLUSH,), jnp.int32),
            pltpu.VMEM((FLUSH,), jnp.int32),
            pltpu.VMEM((FLUSH,), jnp.float32),
            pltpu.VMEM((CAP,), jnp.int32),
            pltpu.VMEM((CAP,), jnp.int32),
            pltpu.VMEM((CAP,), jnp.float32),
            pltpu.VMEM((64, DIM), jnp.float32),
            pltpu.VMEM((64, DIM), jnp.float32),
            pltpu.VMEM((RR_ROWS, DIM), jnp.float32),
            pltpu.VMEM((16,), jnp.int32),
            pltpu.VMEM_SHARED((HALF_NODES, DIM), jnp.float32),
            pltpu.SemaphoreType.DMA,
            pltpu.SemaphoreType.DMA,
            pltpu.SemaphoreType.DMA,
            pltpu.SemaphoreType.DMA,
        ],
        compiler_params=cp,
    )
    y, pcol, prow, pw, ktb = f1(x, colp, rowp, wp)

    f2 = pl.kernel(
        _body2,
        out_type=jax.ShapeDtypeStruct((N_TOTAL, DIM), jnp.float32),
        mesh=mesh,
        scratch_types=[
            pltpu.VMEM((FLUSH,), jnp.int32),
            pltpu.VMEM((FLUSH,), jnp.int32),
            pltpu.VMEM((FLUSH,), jnp.float32),
            pltpu.VMEM((64, DIM), jnp.float32),
            pltpu.VMEM((64, DIM), jnp.float32),
            pltpu.VMEM((RR_ROWS, DIM), jnp.float32),
            pltpu.VMEM((RR_ROWS, DIM), jnp.float32),
            pltpu.VMEM((RR_ROWS, DIM), jnp.float32),
            pltpu.VMEM((16,), jnp.int32),
            pltpu.VMEM_SHARED((HALF_NODES, DIM), jnp.float32),
            pltpu.SemaphoreType.DMA,
            pltpu.SemaphoreType.DMA,
            pltpu.SemaphoreType.DMA,
            pltpu.SemaphoreType.DMA,
        ],
        compiler_params=cp,
    )
    return f2(x, y, pcol, prow, pw, ktb)


def kernel(user_emb, item_emb, edge_index, edge_weight):
    x = jnp.concatenate([user_emb, item_emb], axis=0)
    row = edge_index[0].astype(jnp.int32)
    col = edge_index[1].astype(jnp.int32)
    pad = E_PAD - N_EDGES
    colp = jnp.pad(col, (0, pad)).reshape(N_TILES, EDGES_PER_TILE)
    rowp = jnp.pad(row, (0, pad)).reshape(N_TILES, EDGES_PER_TILE)
    wp = jnp.pad(edge_weight.astype(jnp.float32), (0, pad)).reshape(N_TILES, EDGES_PER_TILE)
    out = _run(x, colp, rowp, wp)
    return out[:N_USERS], out[N_USERS:]


# speedup vs baseline: 1.0126x; 1.0126x over previous
import jax
import jax.numpy as jnp
from jax import lax
from jax.experimental import pallas as pl
from jax.experimental.pallas import tpu as pltpu
from jax.experimental.pallas import tpu_sc as plsc

N_USERS = 25000
N_ITEMS = 25000
N_TOTAL = N_USERS + N_ITEMS
HALF_NODES = N_TOTAL // 2
DIM = 64
N_EDGES = 800000
N_TILES = 16

EDGES_PER_TILE = 50176
E_PAD = N_TILES * EDGES_PER_TILE

FLUSH = 1792
CAP = FLUSH + 16
SCAN_CHUNKS = EDGES_PER_TILE // FLUSH
RBLOCKS = SCAN_CHUNKS + 1
RCAP = RBLOCKS * FLUSH

GROUPS_PER_CHUNK = FLUSH // 128
PAIRS_PER_CHUNK = GROUPS_PER_CHUNK // 2

RR_ROWS = 25
RR_CHUNKS = HALF_NODES // RR_ROWS
RR_ITERS = (RR_CHUNKS + N_TILES - 1) // N_TILES

_GATHER_DNUMS = lax.GatherDimensionNumbers(
    offset_dims=(), collapsed_slice_dims=(0,), start_index_map=(0,))


def _splat(vec16, j):
    idx = jnp.full((16, 1), j, jnp.int32)
    return lax.gather(vec16, idx, _GATHER_DNUMS, slice_sizes=(1,),
                      mode=lax.GatherScatterMode.PROMISE_IN_BOUNDS)


def _axes():
    return lax.axis_index("c"), lax.axis_index("s")


def _fill_zero(zero_v):
    def zf(i, _):
        for h in (0, 16, 32, 48):
            zero_v[i, pl.ds(h, 16)] = jnp.zeros((16,), jnp.float32)
        return 0
    lax.fori_loop(0, RR_ROWS, zf, 0)


def _zero_acc(s, acc, zero_v):
    def zb(k, _):
        ch = s + k * N_TILES
        @pl.when(ch < RR_CHUNKS)
        def _():
            pltpu.sync_copy(zero_v, acc.at[pl.ds(ch * RR_ROWS, RR_ROWS)])
        return 0
    lax.fori_loop(0, RR_ITERS, zb, 0)


def _weight_mul(rows_v, w_c, g):
    def e_body(e16, _):
        w16 = w_c[pl.ds(g * 128 + e16 * 16, 16)]
        for j in range(16):
            wj = _splat(w16, j)
            e = e16 * 16 + j
            for h in (0, 16, 32, 48):
                rows_v[e, pl.ds(h, 16)] = rows_v[e, pl.ds(h, 16)] * wj
        return 0
    lax.fori_loop(0, 8, e_body, 0)


def _do_layer(c, s, src_h, nch, pcol, prow, pw, col_c, row_c, w_c,
              rows_a, rows_b, acc, gsem_a, gsem_b, ssem_a, ssem_b):
    def chunk_body(ch, _):
        off = pl.multiple_of(ch * FLUSH, 8)
        pltpu.sync_copy(pcol.at[c, s, pl.ds(off, FLUSH)], col_c)
        pltpu.sync_copy(prow.at[c, s, pl.ds(off, FLUSH)], row_c)
        pltpu.sync_copy(pw.at[c, s, pl.ds(off, FLUSH)], w_c)

        def pair_body(i, _):
            g0 = i * 2
            dA = pltpu.async_copy(src_h.at[col_c.at[pl.ds(g0 * 128, 128)]], rows_a, gsem_a)
            dB = pltpu.async_copy(src_h.at[col_c.at[pl.ds(g0 * 128 + 128, 128)]], rows_b, gsem_b)
            dA.wait()
            _weight_mul(rows_a, w_c, g0)
            sA = pltpu.async_copy(rows_a, acc.at[row_c.at[pl.ds(g0 * 128, 128)]], ssem_a, add=True)
            dB.wait()
            _weight_mul(rows_b, w_c, g0 + 1)
            sB = pltpu.async_copy(rows_b, acc.at[row_c.at[pl.ds(g0 * 128 + 128, 128)]], ssem_b, add=True)
            sA.wait()
            sB.wait()
            return 0
        lax.fori_loop(0, PAIRS_PER_CHUNK, pair_body, 0)
        return 0
    lax.fori_loop(0, nch, chunk_body, 0)


def _body1(x_hbm, col_hbm, row_hbm, w_hbm, y_hbm, pcol, prow, pw, ktb,
           col_c, row_c, w_c, acol, arow, aw, rows_a, rows_b, zero_v, ktv,
           acc, gsem_a, gsem_b, ssem_a, ssem_b):
    c, s = _axes()
    lo = c * HALF_NODES

    zi = jnp.zeros((16,), jnp.int32)
    zf = jnp.zeros((16,), jnp.float32)

    def chunk_body(ch, carry):
        pltpu.sync_copy(col_hbm.at[s, pl.ds(ch * FLUSH, FLUSH)], col_c)
        pltpu.sync_copy(row_hbm.at[s, pl.ds(ch * FLUSH, FLUSH)], row_c)
        pltpu.sync_copy(w_hbm.at[s, pl.ds(ch * FLUSH, FLUSH)], w_c)

        def v_body(v, carry2):
            fill, op = carry2
            col16 = col_c[pl.ds(v * 16, 16)]
            row16 = row_c[pl.ds(v * 16, 16)]
            w16 = w_c[pl.ds(v * 16, 16)]
            rloc = row16 - lo
            m = (rloc >= 0) & (rloc < HALF_NODES)
            keyv = jnp.where(m, jnp.zeros((16,), jnp.int32),
                             jnp.ones((16,), jnp.int32))
            perm = plsc.sort_key_val(keyv, lax.iota(jnp.int32, 16))[1]
            pidx = jnp.reshape(perm, (16, 1))
            colc = lax.gather(col16, pidx, _GATHER_DNUMS, slice_sizes=(1,),
                              mode=lax.GatherScatterMode.PROMISE_IN_BOUNDS)
            rowc = lax.gather(rloc, pidx, _GATHER_DNUMS, slice_sizes=(1,),
                              mode=lax.GatherScatterMode.PROMISE_IN_BOUNDS)
            wc = lax.gather(w16, pidx, _GATHER_DNUMS, slice_sizes=(1,),
                            mode=lax.GatherScatterMode.PROMISE_IN_BOUNDS)
            acol[pl.ds(fill, 16)] = colc
            arow[pl.ds(fill, 16)] = rowc
            aw[pl.ds(fill, 16)] = wc
            cnt = jnp.max(plsc.all_reduce_population_count(m))
            fill = fill + cnt
            do_flush = fill >= FLUSH

            @pl.when(do_flush)
            def _():
                opa = pl.multiple_of(op, 8)
                pltpu.sync_copy(acol.at[pl.ds(0, FLUSH)], pcol.at[c, s, pl.ds(opa, FLUSH)])
                pltpu.sync_copy(arow.at[pl.ds(0, FLUSH)], prow.at[c, s, pl.ds(opa, FLUSH)])
                pltpu.sync_copy(aw.at[pl.ds(0, FLUSH)], pw.at[c, s, pl.ds(opa, FLUSH)])
                acol[pl.ds(0, 16)] = acol[pl.ds(FLUSH, 16)]
                arow[pl.ds(0, 16)] = arow[pl.ds(FLUSH, 16)]
                aw[pl.ds(0, 16)] = aw[pl.ds(FLUSH, 16)]

            fill = jnp.where(do_flush, fill - FLUSH, fill)
            op = jnp.where(do_flush, op + FLUSH, op)
            return (fill, op)
        return lax.fori_loop(0, FLUSH // 16, v_body, carry)

    fill, op = lax.fori_loop(0, SCAN_CHUNKS, chunk_body,
                             (jnp.int32(0), jnp.int32(0)))

    def z_body(k, _):
        pos = fill + k * 16
        @pl.when(pos < FLUSH)
        def _():
            acol[pl.ds(pos, 16)] = zi
            arow[pl.ds(pos, 16)] = zi
            aw[pl.ds(pos, 16)] = zf
        return 0
    lax.fori_loop(0, FLUSH // 16, z_body, 0)
    opa = pl.multiple_of(op, 8)
    pltpu.sync_copy(acol.at[pl.ds(0, FLUSH)], pcol.at[c, s, pl.ds(opa, FLUSH)])
    pltpu.sync_copy(arow.at[pl.ds(0, FLUSH)], prow.at[c, s, pl.ds(opa, FLUSH)])
    pltpu.sync_copy(aw.at[pl.ds(0, FLUSH)], pw.at[c, s, pl.ds(opa, FLUSH)])
    kt = op + fill
    ktv[pl.ds(0, 16)] = jnp.full((16,), 1, jnp.int32) * kt
    pltpu.sync_copy(ktv, ktb.at[c, s])

    _fill_zero(zero_v)
    _zero_acc(s, acc, zero_v)
    plsc.subcore_barrier()
    nch = lax.div(kt + (FLUSH - 1), FLUSH)
    _do_layer(c, s, x_hbm, nch, pcol, prow, pw, col_c, row_c, w_c,
              rows_a, rows_b, acc, gsem_a, gsem_b, ssem_a, ssem_b)
    plsc.subcore_barrier()

    def yw(k, _):
        ch = s + k * N_TILES
        @pl.when(ch < RR_CHUNKS)
        def _():
            pltpu.sync_copy(acc.at[pl.ds(ch * RR_ROWS, RR_ROWS)],
                            y_hbm.at[pl.ds(lo + ch * RR_ROWS, RR_ROWS)])
        return 0
    lax.fori_loop(0, RR_ITERS, yw, 0)


def _body2(x_hbm, y_hbm, pcol, prow, pw, ktb, out_hbm,
           col_c, row_c, w_c, rows_a, rows_b, zero_v, a_v, b_v, ktv,
           acc, gsem_a, gsem_b, ssem_a, ssem_b):
    c, s = _axes()
    lo = c * HALF_NODES
    c_v = zero_v

    pltpu.sync_copy(ktb.at[c, s], ktv)
    kt = jnp.max(ktv[pl.ds(0, 16)])
    nch = lax.div(kt + (FLUSH - 1), FLUSH)

    _fill_zero(zero_v)
    _zero_acc(s, acc, zero_v)
    plsc.subcore_barrier()
    _do_layer(c, s, y_hbm, nch, pcol, prow, pw, col_c, row_c, w_c,
              rows_a, rows_b, acc, gsem_a, gsem_b, ssem_a, ssem_b)
    plsc.subcore_barrier()

    third = jnp.float32(1.0 / 3.0)

    def epi(k, _):
        ch = s + k * N_TILES
        @pl.when(ch < RR_CHUNKS)
        def _():
            r_loc = ch * RR_ROWS
            r_glob = lo + r_loc
            pltpu.sync_copy(x_hbm.at[pl.ds(r_glob, RR_ROWS)], a_v)
            pltpu.sync_copy(y_hbm.at[pl.ds(r_glob, RR_ROWS)], b_v)
            pltpu.sync_copy(acc.at[pl.ds(r_loc, RR_ROWS)], c_v)

            def erow(r, _):
                for h in (0, 16, 32, 48):
                    a_v[r, pl.ds(h, 16)] = (
                        a_v[r, pl.ds(h, 16)] + b_v[r, pl.ds(h, 16)]
                        + c_v[r, pl.ds(h, 16)]
                    ) * third
                return 0
            lax.fori_loop(0, RR_ROWS, erow, 0)
            pltpu.sync_copy(a_v, out_hbm.at[pl.ds(r_glob, RR_ROWS)])
        return 0
    lax.fori_loop(0, RR_ITERS, epi, 0)


@jax.jit
def _run(x, colp, rowp, wp):
    mesh = plsc.VectorSubcoreMesh(core_axis_name="c", subcore_axis_name="s")
    cp = pltpu.CompilerParams(use_tc_tiling_on_sc=False, needs_layout_passes=False)
    f1 = pl.kernel(
        _body1,
        out_type=(
            jax.ShapeDtypeStruct((N_TOTAL, DIM), jnp.float32),
            jax.ShapeDtypeStruct((2, N_TILES, RCAP), jnp.int32),
            jax.ShapeDtypeStruct((2, N_TILES, RCAP), jnp.int32),
            jax.ShapeDtypeStruct((2, N_TILES, RCAP), jnp.float32),
            jax.ShapeDtypeStruct((2, N_TILES, 16), jnp.int32),
        ),
        mesh=mesh,
        scratch_types=[
            pltpu.VMEM((FLUSH,), jnp.int32),
            pltpu.VMEM((FLUSH,), jnp.int32),
            pltpu.VMEM((FLUSH,), jnp.float32),
            pltpu.VMEM((CAP,), jnp.int32),
            pltpu.VMEM((CAP,), jnp.int32),
            pltpu.VMEM((CAP,), jnp.float32),
            pltpu.VMEM((128, DIM), jnp.float32),
            pltpu.VMEM((128, DIM), jnp.float32),
            pltpu.VMEM((RR_ROWS, DIM), jnp.float32),
            pltpu.VMEM((16,), jnp.int32),
            pltpu.VMEM_SHARED((HALF_NODES, DIM), jnp.float32),
            pltpu.SemaphoreType.DMA,
            pltpu.SemaphoreType.DMA,
            pltpu.SemaphoreType.DMA,
            pltpu.SemaphoreType.DMA,
        ],
        compiler_params=cp,
    )
    y, pcol, prow, pw, ktb = f1(x, colp, rowp, wp)

    f2 = pl.kernel(
        _body2,
        out_type=jax.ShapeDtypeStruct((N_TOTAL, DIM), jnp.float32),
        mesh=mesh,
        scratch_types=[
            pltpu.VMEM((FLUSH,), jnp.int32),
            pltpu.VMEM((FLUSH,), jnp.int32),
            pltpu.VMEM((FLUSH,), jnp.float32),
            pltpu.VMEM((128, DIM), jnp.float32),
            pltpu.VMEM((128, DIM), jnp.float32),
            pltpu.VMEM((RR_ROWS, DIM), jnp.float32),
            pltpu.VMEM((RR_ROWS, DIM), jnp.float32),
            pltpu.VMEM((RR_ROWS, DIM), jnp.float32),
            pltpu.VMEM((16,), jnp.int32),
            pltpu.VMEM_SHARED((HALF_NODES, DIM), jnp.float32),
            pltpu.SemaphoreType.DMA,
            pltpu.SemaphoreType.DMA,
            pltpu.SemaphoreType.DMA,
            pltpu.SemaphoreType.DMA,
        ],
        compiler_params=cp,
    )
    return f2(x, y, pcol, prow, pw, ktb)


def kernel(user_emb, item_emb, edge_index, edge_weight):
    x = jnp.concatenate([user_emb, item_emb], axis=0)
    row = edge_index[0].astype(jnp.int32)
    col = edge_index[1].astype(jnp.int32)
    pad = E_PAD - N_EDGES
    colp = jnp.pad(col, (0, pad)).reshape(N_TILES, EDGES_PER_TILE)
    rowp = jnp.pad(row, (0, pad)).reshape(N_TILES, EDGES_PER_TILE)
    wp = jnp.pad(edge_weight.astype(jnp.float32), (0, pad)).reshape(N_TILES, EDGES_PER_TILE)
    out = _run(x, colp, rowp, wp)
    return out[:N_USERS], out[N_USERS:]


# speedup vs baseline: 2.5397x; 2.5081x over previous
import jax
import jax.numpy as jnp
from jax import lax
from jax.experimental import pallas as pl
from jax.experimental.pallas import tpu as pltpu
from jax.experimental.pallas import tpu_sc as plsc

N_USERS = 25000
N_ITEMS = 25000
N_TOTAL = N_USERS + N_ITEMS
DIM = 64
HALF = 32
N_EDGES = 800000
N_TILES = 16

EDGES_PER_TILE = 50176
E_PAD = N_TILES * EDGES_PER_TILE
GROUPS_PER_TILE = EDGES_PER_TILE // 128
GROUPS_PER_CHUNK = 28
N_CHUNKS = GROUPS_PER_TILE // GROUPS_PER_CHUNK
EDGES_PER_CHUNK = GROUPS_PER_CHUNK * 128

ROWS_PER_TILE = N_TOTAL // N_TILES
EPI_ROWS = 125
EPI_CHUNKS = ROWS_PER_TILE // EPI_ROWS

_GATHER_DNUMS = lax.GatherDimensionNumbers(
    offset_dims=(), collapsed_slice_dims=(0,), start_index_map=(0,))


def _splat(vec16, j):
    idx = jnp.full((16, 1), j, jnp.int32)
    return lax.gather(vec16, idx, _GATHER_DNUMS, slice_sizes=(1,),
                      mode=lax.GatherScatterMode.PROMISE_IN_BOUNDS)


def _body(xs_hbm, col_hbm, row_hbm, w_hbm, y_hbm, out_hbm,
          col_v, row_v, w_v, rows_a, rows_b, zero_v, a_v, b_v, acc,
          sem, gsem_a, gsem_b, ssem_a, ssem_b):
    c_v = zero_v
    c = lax.axis_index("c")
    s = lax.axis_index("s")
    x_h = xs_hbm.at[c]
    y_h = y_hbm.at[c]
    o_h = out_hbm.at[c]

    def zfill(i, _):
        z = jnp.zeros((16,), jnp.float32)
        zero_v[i, pl.ds(0, 16)] = z
        zero_v[i, pl.ds(16, 16)] = z
        return 0
    lax.fori_loop(0, EPI_ROWS, zfill, 0)

    def zero_acc():
        def zb(i, _):
            pltpu.sync_copy(zero_v, acc.at[pl.ds(s * ROWS_PER_TILE + i * EPI_ROWS, EPI_ROWS)])
            return 0
        lax.fori_loop(0, EPI_CHUNKS, zb, 0)

    def weight_mul(rows_v, g):
        def e_body(e16, _):
            w16 = w_v[pl.ds(g * 128 + e16 * 16, 16)]
            for j in range(16):
                wj = _splat(w16, j)
                e = e16 * 16 + j
                rows_v[e, pl.ds(0, 16)] = rows_v[e, pl.ds(0, 16)] * wj
                rows_v[e, pl.ds(16, 16)] = rows_v[e, pl.ds(16, 16)] * wj
            return 0
        lax.fori_loop(0, 8, e_body, 0)

    def do_layer(src_h):
        def chunk_body(ch, _):
            pltpu.sync_copy(col_hbm.at[s, pl.ds(ch * GROUPS_PER_CHUNK, GROUPS_PER_CHUNK)], col_v)
            pltpu.sync_copy(row_hbm.at[s, pl.ds(ch * GROUPS_PER_CHUNK, GROUPS_PER_CHUNK)], row_v)
            pltpu.sync_copy(w_hbm.at[s, pl.ds(ch * EDGES_PER_CHUNK, EDGES_PER_CHUNK)], w_v)

            def pair_body(i, _):
                g0 = i * 2
                dA = pltpu.async_copy(src_h.at[col_v.at[g0]], rows_a, gsem_a)
                dB = pltpu.async_copy(src_h.at[col_v.at[g0 + 1]], rows_b, gsem_b)
                dA.wait()
                weight_mul(rows_a, g0)
                sA = pltpu.async_copy(rows_a, acc.at[row_v.at[g0]], ssem_a, add=True)
                dB.wait()
                weight_mul(rows_b, g0 + 1)
                sB = pltpu.async_copy(rows_b, acc.at[row_v.at[g0 + 1]], ssem_b, add=True)
                sA.wait()
                sB.wait()
                return 0
            lax.fori_loop(0, GROUPS_PER_CHUNK // 2, pair_body, 0)
            return 0
        lax.fori_loop(0, N_CHUNKS, chunk_body, 0)

    zero_acc()
    plsc.subcore_barrier()
    do_layer(x_h)
    plsc.subcore_barrier()

    def y_copy(i, _):
        r0 = s * ROWS_PER_TILE + i * EPI_ROWS
        pltpu.sync_copy(acc.at[pl.ds(r0, EPI_ROWS)], y_h.at[pl.ds(r0, EPI_ROWS)])
        return 0
    lax.fori_loop(0, EPI_CHUNKS, y_copy, 0)
    zero_acc()
    plsc.subcore_barrier()
    do_layer(y_h)
    plsc.subcore_barrier()

    third = jnp.float32(1.0 / 3.0)

    def epi(i, _):
        r0 = s * ROWS_PER_TILE + i * EPI_ROWS
        pltpu.sync_copy(x_h.at[pl.ds(r0, EPI_ROWS)], a_v)
        pltpu.sync_copy(y_h.at[pl.ds(r0, EPI_ROWS)], b_v)
        pltpu.sync_copy(acc.at[pl.ds(r0, EPI_ROWS)], c_v)

        def erow(r, _):
            for h in (0, 16):
                a_v[r, pl.ds(h, 16)] = (
                    a_v[r, pl.ds(h, 16)] + b_v[r, pl.ds(h, 16)] + c_v[r, pl.ds(h, 16)]
                ) * third
            return 0
        lax.fori_loop(0, EPI_ROWS, erow, 0)
        pltpu.sync_copy(a_v, o_h.at[pl.ds(r0, EPI_ROWS)])
        return 0
    lax.fori_loop(0, EPI_CHUNKS, epi, 0)


@jax.jit
def _run(xs, colp, rowp, wp):
    mesh = plsc.VectorSubcoreMesh(core_axis_name="c", subcore_axis_name="s")
    f = pl.kernel(
        _body,
        out_type=(
            jax.ShapeDtypeStruct((2, N_TOTAL, HALF), jnp.float32),
            jax.ShapeDtypeStruct((2, N_TOTAL, HALF), jnp.float32),
        ),
        mesh=mesh,
        scratch_types=[
            pltpu.VMEM((GROUPS_PER_CHUNK, 128), jnp.int32),
            pltpu.VMEM((GROUPS_PER_CHUNK, 128), jnp.int32),
            pltpu.VMEM((EDGES_PER_CHUNK,), jnp.float32),
            pltpu.VMEM((128, HALF), jnp.float32),
            pltpu.VMEM((128, HALF), jnp.float32),
            pltpu.VMEM((EPI_ROWS, HALF), jnp.float32),
            pltpu.VMEM((EPI_ROWS, HALF), jnp.float32),
            pltpu.VMEM((EPI_ROWS, HALF), jnp.float32),
            pltpu.VMEM_SHARED((N_TOTAL, HALF), jnp.float32),
            pltpu.SemaphoreType.DMA,
            pltpu.SemaphoreType.DMA,
            pltpu.SemaphoreType.DMA,
            pltpu.SemaphoreType.DMA,
            pltpu.SemaphoreType.DMA,
        ],
        compiler_params=pltpu.CompilerParams(use_tc_tiling_on_sc=False),
    )
    return f(xs, colp, rowp, wp)


def kernel(user_emb, item_emb, edge_index, edge_weight):
    x = jnp.concatenate([user_emb, item_emb], axis=0)
    xs = jnp.stack([x[:, :HALF], x[:, HALF:]])
    row = edge_index[0].astype(jnp.int32)
    col = edge_index[1].astype(jnp.int32)
    pad = E_PAD - N_EDGES
    colp = jnp.pad(col, (0, pad)).reshape(N_TILES, GROUPS_PER_TILE, 128)
    rowp = jnp.pad(row, (0, pad)).reshape(N_TILES, GROUPS_PER_TILE, 128)
    wp = jnp.pad(edge_weight.astype(jnp.float32), (0, pad)).reshape(N_TILES, EDGES_PER_TILE)
    _y, out = _run(xs, colp, rowp, wp)
    xf = jnp.concatenate([out[0], out[1]], axis=1)
    return xf[:N_USERS], xf[N_USERS:]
